# Initial kernel scaffold; baseline (speedup 1.0000x reference)
import jax
import jax.numpy as jnp
from jax.experimental import pallas as pl
from jax.experimental.pallas import tpu as pltpu

K = 128
NUM_LAYERS = 4
DISCARD = 2


def _gcn(x, src, dst, W, b, n):
    ones = jnp.ones(src.shape[0], dtype=x.dtype)
    deg_out = jnp.clip(jax.ops.segment_sum(ones, src, num_segments=n), 1.0)
    deg_in = jnp.clip(jax.ops.segment_sum(ones, dst, num_segments=n), 1.0)
    norm = jax.lax.rsqrt(deg_out[src] * deg_in[dst])
    msg = x[src] * norm[:, None]
    agg = jax.ops.segment_sum(msg, dst, num_segments=n)
    return agg @ W + b


def _head_body(stu_ref, exer_ref, concept_ref, edisc_ref, kn_ref,
               fsW_ref, fsb_ref, feW_ref, feb_ref,
               p1W_ref, p1b_ref, p2W_ref, p2b_ref, p3W_ref, p3b_ref,
               out_ref):
    stu = stu_ref[...]
    exer = exer_ref[...]
    concept = concept_ref[...]
    fsW = fsW_ref[...]
    feW = feW_ref[...]
    pu = stu @ fsW[0, :K]
    kv1 = concept @ fsW[0, K:]
    eu = exer @ feW[0, :K]
    kv2 = concept @ feW[0, K:]
    prof = jax.nn.sigmoid(pu[:, None] + kv1[None, :] + fsb_ref[0])
    diff = jax.nn.sigmoid(eu[:, None] + kv2[None, :] + feb_ref[0])
    edisc = jax.nn.sigmoid(edisc_ref[...]) * 10.0
    x = edisc * (prof - diff) * kn_ref[...]
    x = jax.nn.sigmoid(
        jax.lax.dot_general(x, p1W_ref[...], (((1,), (1,)), ((), ()))) + p1b_ref[...])
    x = jax.nn.sigmoid(
        jax.lax.dot_general(x, p2W_ref[...], (((1,), (1,)), ((), ()))) + p2b_ref[...])
    out_ref[...] = jax.nn.sigmoid(
        jax.lax.dot_general(x, p3W_ref[...], (((1,), (1,)), ((), ()))) + p3b_ref[...])


def _head(stu_emb, exer_emb, concept, edisc, kn_emb,
          fs_W, fs_b, fe_W, fe_b, p1_W, p1_b, p2_W, p2_b, p3_W, p3_b):
    B = stu_emb.shape[0]
    return pl.pallas_call(
        _head_body,
        out_shape=jax.ShapeDtypeStruct((B, 1), jnp.float32),
    )(stu_emb, exer_emb, concept, edisc, kn_emb,
      fs_W, fs_b, fe_W, fe_b, p1_W, p1_b, p2_W, p2_b, p3_W, p3_b)


def kernel(stu_id, exer_id, kn_emb, input_knowedge_ids, history, entity, e_disc,
           src_sim, dst_sim, src_pre, dst_pre, src_ec, dst_ec,
           W_sim, b_sim, W_pre, b_pre, W_ec, b_ec,
           fs_W, fs_b, fe_W, fe_b, p1_W, p1_b, p2_W, p2_b, p3_W, p3_b):
    N = entity.shape[0]
    embs = [entity]
    for l in range(NUM_LAYERS):
        h = embs[-1]
        s = _gcn(h, src_sim, dst_sim, W_sim[l], b_sim[l], N)
        p = _gcn(h, src_pre, dst_pre, W_pre[l], b_pre[l], N)
        e = _gcn(h, src_ec, dst_ec, W_ec[l], b_ec[l], N)
        embs.append(s + p + e)
    full = jnp.mean(jnp.stack(embs), axis=0)
    disc_emb = jnp.mean(jnp.stack(embs[DISCARD + 1:]), axis=0)
    concept_entity = full[:K]
    exer_entity = full[K:]
    stu_exe_entity = disc_emb[K:]
    stu_emb = jnp.mean(stu_exe_entity[history], axis=1)
    exer_emb = exer_entity[exer_id]
    return _head(stu_emb, exer_emb, concept_entity, e_disc[exer_id], kn_emb,
                 fs_W, fs_b, fe_W, fe_b, p1_W, p1_b, p2_W, p2_b, p3_W, p3_b)


# speedup vs baseline: 1.0021x; 1.0021x over previous
import jax
import jax.numpy as jnp
from jax.experimental import pallas as pl
from jax.experimental.pallas import tpu as pltpu

K = 128
NUM_LAYERS = 4
DISCARD = 2


def _gcn(x, src, dst, W, b, n):
    ones = jnp.ones(src.shape[0], dtype=x.dtype)
    deg_out = jnp.clip(jax.ops.segment_sum(ones, src, num_segments=n), 1.0)
    deg_in = jnp.clip(jax.ops.segment_sum(ones, dst, num_segments=n), 1.0)
    norm = jax.lax.rsqrt(deg_out[src] * deg_in[dst])
    msg = x[src] * norm[:, None]
    agg = jax.ops.segment_sum(msg, dst, num_segments=n)
    return agg @ W + b


def _head_body(stu_ref, exer_ref, concept_ref, edisc_ref, kn_ref,
               fsW_ref, fsb_ref, feW_ref, feb_ref,
               p1W_ref, p1b_ref, p2W_ref, p2b_ref, p3W_ref, p3b_ref,
               out_ref):
    stu = stu_ref[...]
    exer = exer_ref[...]
    concept = concept_ref[...]
    fsW = fsW_ref[...]
    feW = feW_ref[...]
    pu = jnp.sum(stu * fsW[:, :K], axis=1, keepdims=True)
    eu = jnp.sum(exer * feW[:, :K], axis=1, keepdims=True)
    kv1 = jax.lax.dot_general(fsW[:, K:], concept, (((1,), (1,)), ((), ())))
    kv2 = jax.lax.dot_general(feW[:, K:], concept, (((1,), (1,)), ((), ())))
    prof = jax.nn.sigmoid(pu + kv1 + fsb_ref[0])
    diff = jax.nn.sigmoid(eu + kv2 + feb_ref[0])
    edisc = jax.nn.sigmoid(edisc_ref[...]) * 10.0
    x = edisc * (prof - diff) * kn_ref[...]
    x = jax.nn.sigmoid(
        jax.lax.dot_general(x, p1W_ref[...], (((1,), (1,)), ((), ()))) + p1b_ref[...][None, :])
    x = jax.nn.sigmoid(
        jax.lax.dot_general(x, p2W_ref[...], (((1,), (1,)), ((), ()))) + p2b_ref[...][None, :])
    y = jnp.sum(x * p3W_ref[...], axis=1, keepdims=True) + p3b_ref[0]
    out_ref[...] = jax.nn.sigmoid(y)


def _head(stu_emb, exer_emb, concept, edisc, kn_emb,
          fs_W, fs_b, fe_W, fe_b, p1_W, p1_b, p2_W, p2_b, p3_W, p3_b):
    B = stu_emb.shape[0]
    return pl.pallas_call(
        _head_body,
        out_shape=jax.ShapeDtypeStruct((B, 1), jnp.float32),
    )(stu_emb, exer_emb, concept, edisc, kn_emb,
      fs_W, fs_b, fe_W, fe_b, p1_W, p1_b, p2_W, p2_b, p3_W, p3_b)


def kernel(stu_id, exer_id, kn_emb, input_knowedge_ids, history, entity, e_disc,
           src_sim, dst_sim, src_pre, dst_pre, src_ec, dst_ec,
           W_sim, b_sim, W_pre, b_pre, W_ec, b_ec,
           fs_W, fs_b, fe_W, fe_b, p1_W, p1_b, p2_W, p2_b, p3_W, p3_b):
    N = entity.shape[0]
    embs = [entity]
    for l in range(NUM_LAYERS):
        h = embs[-1]
        s = _gcn(h, src_sim, dst_sim, W_sim[l], b_sim[l], N)
        p = _gcn(h, src_pre, dst_pre, W_pre[l], b_pre[l], N)
        e = _gcn(h, src_ec, dst_ec, W_ec[l], b_ec[l], N)
        embs.append(s + p + e)
    full = jnp.mean(jnp.stack(embs), axis=0)
    disc_emb = jnp.mean(jnp.stack(embs[DISCARD + 1:]), axis=0)
    concept_entity = full[:K]
    exer_entity = full[K:]
    stu_exe_entity = disc_emb[K:]
    stu_emb = jnp.mean(stu_exe_entity[history], axis=1)
    exer_emb = exer_entity[exer_id]
    return _head(stu_emb, exer_emb, concept_entity, e_disc[exer_id], kn_emb,
                 fs_W, fs_b, fe_W, fe_b, p1_W, p1_b, p2_W, p2_b, p3_W, p3_b)


# speedup vs baseline: 2.2594x; 2.2546x over previous
import functools

import jax
import jax.numpy as jnp
from jax import lax
from jax.experimental import pallas as pl
from jax.experimental.pallas import tpu as pltpu
import jax.experimental.pallas.tpu_sc as plsc

NC = 2
NS = 16
LN = 16
IC = 128
WV = 4


def _mesh():
    return plsc.VectorSubcoreMesh(
        core_axis_name="c", subcore_axis_name="s", num_cores=NC, num_subcores=NS)


def _deg_call(deg_idxR, NP, HN, HNA, CPT):
    ZR = HNA // NS
    RO = HN // NS
    SUB = 64
    NW8 = (CPT * IC) // (8 * SUB)
    assert ZR % 32 == 0 and (CPT * IC) % (8 * SUB) == 0

    @functools.partial(
        pl.kernel,
        out_type=jax.ShapeDtypeStruct((6, NP, 128), jnp.float32),
        mesh=_mesh(),
        scratch_types=[
            pltpu.VMEM_SHARED((HNA, 128), jnp.float32),
            pltpu.VMEM((32, 128), jnp.float32),
            pltpu.VMEM((SUB, 128), jnp.float32),
            pltpu.VMEM((8, 1, SUB), jnp.int32),
            pltpu.SemaphoreType.DMA,
        ],
    )
    def k(idx_hbm, deg_out, acc, zbuf, ones, ib, ssem):
        c = lax.axis_index("c")
        s = lax.axis_index("s")

        @pl.loop(0, 32)
        def _(i):
            for m in range(8):
                zbuf[i, pl.ds(m * LN, LN)] = jnp.zeros((LN,), jnp.float32)

        @pl.loop(0, SUB)
        def _(i):
            for m in range(8):
                ones[i, pl.ds(m * LN, LN)] = jnp.ones((LN,), jnp.float32)

        base = s * (CPT * IC)
        for aid in range(6):
            for z in range(ZR // 32):
                pltpu.sync_copy(zbuf, acc.at[pl.ds(s * ZR + z * 32, 32)])
            plsc.subcore_barrier()

            @pl.loop(0, NW8)
            def _(w, aid=aid):
                st = base + w * (8 * SUB)
                for t in range(8):
                    pltpu.sync_copy(
                        idx_hbm.at[c, aid, 0, pl.ds(st + t * SUB, SUB)],
                        ib.at[t, 0])
                for t in range(8):
                    pltpu.async_copy(ones, acc.at[ib.at[t, 0]], ssem,
                                     add=True)
                for t in range(8):
                    pltpu.make_async_copy(ones, acc.at[ib.at[t, 0]],
                                          ssem).wait()

            plsc.subcore_barrier()
            pltpu.sync_copy(acc.at[pl.ds(s * RO, RO)],
                            deg_out.at[aid, pl.ds(c * HN + s * RO, RO)])
            plsc.subcore_barrier()

    return k(deg_idxR)


def _agg_call(hs0, hs1, hs2, srcp, dstR, NP, HN, HNA, CPT):
    AW = 1
    ZR = HNA // NS
    RO = HN // NS
    NWV = CPT // AW
    assert NWV >= 4 and NWV % 2 == 0 and ZR % IC == 0

    @functools.partial(
        pl.kernel,
        out_type=jax.ShapeDtypeStruct((3, NP, 128), jnp.float32),
        mesh=_mesh(),
        scratch_types=[
            pltpu.VMEM_SHARED((HNA, 128), jnp.float32),
            pltpu.VMEM((IC // 2, 128), jnp.float32),
            pltpu.VMEM((2 * AW, 1, IC), jnp.int32),
            pltpu.VMEM((2 * AW, 1, IC), jnp.int32),
            pltpu.VMEM((2 * AW, IC, 128), jnp.float32),
            pltpu.SemaphoreType.DMA,
            pltpu.SemaphoreType.DMA,
            pltpu.SemaphoreType.DMA,
            pltpu.SemaphoreType.DMA,
        ],
    )
    def k(t0, t1, t2, src_h, dstR_h, agg_out,
          acc, zbuf, sidx, didx, rowb, gsemA, gsemB, ssemA, ssemB):
        c = lax.axis_index("c")
        s = lax.axis_index("s")
        gsems = (gsemA, gsemB)
        ssems = (ssemA, ssemB)

        @pl.loop(0, IC // 2)
        def _(i):
            for m in range(8):
                zbuf[i, pl.ds(m * LN, LN)] = jnp.zeros((LN,), jnp.float32)

        ebase = s * (CPT * IC)
        for g, tbl in enumerate((t0, t1, t2)):

            def load_and_gather(w, P):
                st = ebase + w * (AW * IC)
                o = P * AW
                for t in range(AW):
                    pltpu.sync_copy(src_h.at[g, 0, pl.ds(st + t * IC, IC)],
                                    sidx.at[o + t, 0])
                    pltpu.sync_copy(
                        dstR_h.at[c, g, 0, pl.ds(st + t * IC, IC)],
                        didx.at[o + t, 0])
                for t in range(AW):
                    pltpu.async_copy(tbl.at[sidx.at[o + t, 0]],
                                     rowb.at[o + t], gsems[P])

            def wait_gather(P):
                o = P * AW
                for t in range(AW):
                    pltpu.make_async_copy(tbl.at[sidx.at[o + t, 0]],
                                          rowb.at[o + t], gsems[P]).wait()

            def start_scatter(P):
                o = P * AW
                for t in range(AW):
                    pltpu.async_copy(rowb.at[o + t],
                                     acc.at[didx.at[o + t, 0]],
                                     ssems[P], add=True)

            def wait_scatter(P):
                o = P * AW
                for t in range(AW):
                    pltpu.make_async_copy(rowb.at[o + t],
                                          acc.at[didx.at[o + t, 0]],
                                          ssems[P]).wait()

            for z in range(2 * (ZR // IC)):
                pltpu.sync_copy(
                    zbuf, acc.at[pl.ds(s * ZR + z * (IC // 2), IC // 2)])
            plsc.subcore_barrier()

            load_and_gather(0, 0)
            load_and_gather(1, 1)
            wait_gather(0)
            start_scatter(0)

            @pl.loop(0, (NWV - 2) // 2)
            def _(i):
                for pp, P in ((0, 1), (1, 0)):
                    w = 1 + i * 2 + pp
                    wait_scatter(1 - P)
                    load_and_gather(w + 1, 1 - P)
                    wait_gather(P)
                    start_scatter(P)

            wait_scatter(0)
            wait_gather(1)
            start_scatter(1)
            wait_scatter(1)

            plsc.subcore_barrier()
            pltpu.sync_copy(acc.at[pl.ds(s * RO, RO)],
                            agg_out.at[g, pl.ds(c * HN + s * RO, RO)])
            plsc.subcore_barrier()

    return k(hs0, hs1, hs2, srcp, dstR)


def _gath_call(disc, full, histp, eshift, eraw, edtab, B, HH):
    SPW = B // (NC * NS)
    HP = histp.shape[-1]

    @functools.partial(
        pl.kernel,
        out_type=(jax.ShapeDtypeStruct((B, 128), jnp.float32),
                  jax.ShapeDtypeStruct((B, 128), jnp.float32),
                  jax.ShapeDtypeStruct((B, 128), jnp.float32)),
        mesh=_mesh(),
        scratch_types=[
            pltpu.VMEM((16, 1, HP), jnp.int32),
            pltpu.VMEM((2, HP, 128), jnp.float32),
            pltpu.VMEM((SPW, 128), jnp.float32),
            pltpu.VMEM((SPW,), jnp.int32),
            pltpu.VMEM((SPW,), jnp.int32),
            pltpu.SemaphoreType.DMA,
            pltpu.SemaphoreType.DMA,
            pltpu.SemaphoreType.DMA,
        ],
    )
    def k(disc_h, full_h, histp_h, eshift_h, eraw_h, edtab_h,
          stu_o, exe_o, edc_o,
          hblk, hrows, stl, eidx, edi, hsemA, hsemB, esem):
        c = lax.axis_index("c")
        s = lax.axis_index("s")
        wid = s * NC + c
        b0 = wid * SPW
        hsems = (hsemA, hsemB)

        pltpu.sync_copy(eshift_h.at[pl.ds(b0, SPW)], eidx)
        pltpu.async_copy(full_h.at[eidx], stl, esem).wait()
        pltpu.sync_copy(stl, exe_o.at[pl.ds(b0, SPW)])
        pltpu.sync_copy(eraw_h.at[pl.ds(b0, SPW)], edi)
        pltpu.async_copy(edtab_h.at[edi], hrows.at[0, pl.ds(0, SPW)],
                         esem).wait()
        pltpu.sync_copy(hrows.at[0, pl.ds(0, SPW)], edc_o.at[pl.ds(b0, SPW)])

        def hrow(j):
            return (lax.shift_right_logical(j, 3) & 1) * 8 + (j & 7)

        def start_hist(j, P):
            pltpu.async_copy(disc_h.at[hblk.at[hrow(j), 0]], hrows.at[P],
                             hsems[P])

        def wait_hist(j, P):
            pltpu.make_async_copy(disc_h.at[hblk.at[hrow(j), 0]],
                                  hrows.at[P], hsems[P]).wait()

        pltpu.sync_copy(histp_h.at[pl.ds(b0, 8)], hblk.at[pl.ds(0, 8)])
        start_hist(0, 0)

        @pl.loop(0, SPW // 2)
        def _(i):
            for pp in range(2):
                j = i * 2 + pp
                nj = j + 1

                @pl.when(jnp.logical_and((nj & 7) == 0, nj < SPW))
                def _():
                    o = (lax.shift_right_logical(nj, 3) & 1) * 8
                    st = pl.multiple_of(b0 + nj, 8)
                    pltpu.sync_copy(histp_h.at[pl.ds(st, 8)],
                                    hblk.at[pl.ds(o, 8)])

                @pl.when(nj < SPW)
                def _():
                    start_hist(nj, 1 - pp)

                wait_hist(j, pp)
                hr = hrows.at[pp]
                for m in range(8):
                    a = jnp.zeros((LN,), jnp.float32)
                    for t in range(HH):
                        a = a + hr[t, pl.ds(m * LN, LN)]
                    stl[j, pl.ds(m * LN, LN)] = a * (1.0 / HH)

        pltpu.sync_copy(stl, stu_o.at[pl.ds(b0, SPW)])

    return k(disc, full, histp, eshift, eraw, edtab)


def _prep_call(deg, entity, NP, R):
    N = entity.shape[0]
    G = N // R

    def body(deg_ref, ent_ref, rout_ref, rin_ref, h0_ref, h1_ref, h2_ref):
        d = deg_ref[...][:, :, 0:1]
        r = lax.rsqrt(jnp.maximum(d, 1.0))
        rout_ref[...] = r[0:3]
        rin_ref[...] = r[3:6]
        ent = ent_ref[...]
        h0_ref[...] = ent * r[0]
        h1_ref[...] = ent * r[1]
        h2_ref[...] = ent * r[2]

    hs_sds = jax.ShapeDtypeStruct((NP, 128), jnp.float32)
    hs_spec = pl.BlockSpec((R, 128), lambda i: (i, 0))
    return pl.pallas_call(
        body,
        grid=(G,),
        in_specs=[pl.BlockSpec((6, R, 128), lambda i: (0, i, 0)),
                  pl.BlockSpec((R, 128), lambda i: (i, 0))],
        out_specs=[pl.BlockSpec((3, R, 1), lambda i: (0, i, 0)),
                   pl.BlockSpec((3, R, 1), lambda i: (0, i, 0)),
                   hs_spec, hs_spec, hs_spec],
        out_shape=[jax.ShapeDtypeStruct((3, N, 1), jnp.float32),
                   jax.ShapeDtypeStruct((3, N, 1), jnp.float32),
                   hs_sds, hs_sds, hs_sds],
    )(deg, entity)


def _layer_call(l, agg, rin, rout, Wst, bs, facc, dacc, NP, R):
    N = facc.shape[0]
    G = N // R

    def body(agg_ref, rin_ref, W_ref, bs_ref, facc_ref, *rest):
        if l < 4:
            rout_ref = rest[0]
            outs = rest[1:]
        else:
            dacc_ref = rest[0]
            outs = rest[1:]
        x = agg_ref[...]
        rin_v = rin_ref[...]
        W = W_ref[...]
        acc = jnp.zeros((R, 128), jnp.float32)
        for g in range(3):
            xg = x[g] * rin_v[g]
            acc = acc + jnp.dot(xg, W[g], preferred_element_type=jnp.float32)
        h = acc + bs_ref[...]
        if l < 4:
            ro = rout_ref[...]
            outs[0][...] = h * ro[0]
            outs[1][...] = h * ro[1]
            outs[2][...] = h * ro[2]
            outs[3][...] = facc_ref[...] + h
            if l == 3:
                outs[4][...] = h
        else:
            outs[0][...] = (facc_ref[...] + h) * 0.2
            outs[1][...] = (dacc_ref[...] + h) * 0.5

    row_spec = pl.BlockSpec((R, 128), lambda i: (i, 0))
    r3_spec = pl.BlockSpec((3, R, 1), lambda i: (0, i, 0))
    in_specs = [pl.BlockSpec((3, R, 128), lambda i: (0, i, 0)),
                r3_spec,
                pl.BlockSpec((3, 128, 128), lambda i: (0, 0, 0)),
                pl.BlockSpec((1, 128), lambda i: (0, 0)),
                row_spec]
    ins = [agg, rin, Wst, bs, facc]
    row_sds = jax.ShapeDtypeStruct((N, 128), jnp.float32)
    hs_sds = jax.ShapeDtypeStruct((NP, 128), jnp.float32)
    if l < 4:
        in_specs.append(r3_spec)
        ins.append(rout)
        out_shape = [hs_sds, hs_sds, hs_sds, row_sds]
        out_specs = [row_spec, row_spec, row_spec, row_spec]
        if l == 3:
            out_shape.append(row_sds)
            out_specs.append(row_spec)
    else:
        in_specs.append(row_spec)
        ins.append(dacc)
        out_shape = [row_sds, row_sds]
        out_specs = [row_spec, row_spec]
    return pl.pallas_call(
        body, grid=(G,), in_specs=in_specs,
        out_specs=out_specs, out_shape=out_shape,
    )(*ins)


def _head_body(stu_ref, exer_ref, concept_ref, edisc_ref, kn_ref,
               fsW_ref, fsb_ref, feW_ref, feb_ref,
               p1W_ref, p1b_ref, p2W_ref, p2b_ref, p3W_ref, p3b_ref,
               out_ref):
    K = 128
    stu = stu_ref[...]
    exer = exer_ref[...]
    concept = concept_ref[...]
    fsW = fsW_ref[...]
    feW = feW_ref[...]
    pu = jnp.sum(stu * fsW[:, :K], axis=1, keepdims=True)
    eu = jnp.sum(exer * feW[:, :K], axis=1, keepdims=True)
    kv1 = jax.lax.dot_general(fsW[:, K:], concept, (((1,), (1,)), ((), ())))
    kv2 = jax.lax.dot_general(feW[:, K:], concept, (((1,), (1,)), ((), ())))
    prof = jax.nn.sigmoid(pu + kv1 + fsb_ref[0])
    diff = jax.nn.sigmoid(eu + kv2 + feb_ref[0])
    edisc = jax.nn.sigmoid(edisc_ref[...]) * 10.0
    x = edisc * (prof - diff) * kn_ref[...]
    x = jax.nn.sigmoid(
        jax.lax.dot_general(x, p1W_ref[...], (((1,), (1,)), ((), ())))
        + p1b_ref[...][None, :])
    x = jax.nn.sigmoid(
        jax.lax.dot_general(x, p2W_ref[...], (((1,), (1,)), ((), ())))
        + p2b_ref[...][None, :])
    y = jnp.sum(x * p3W_ref[...], axis=1, keepdims=True) + p3b_ref[0]
    out_ref[...] = jax.nn.sigmoid(y)


def _head(stu_emb, exer_emb, concept, edisc, kn_emb,
          fs_W, fs_b, fe_W, fe_b, p1_W, p1_b, p2_W, p2_b, p3_W, p3_b):
    B = stu_emb.shape[0]
    return pl.pallas_call(
        _head_body,
        out_shape=jax.ShapeDtypeStruct((B, 1), jnp.float32),
    )(stu_emb, exer_emb, concept, edisc, kn_emb,
      fs_W, fs_b, fe_W, fe_b, p1_W, p1_b, p2_W, p2_b, p3_W, p3_b)


def kernel(stu_id, exer_id, kn_emb, input_knowedge_ids, history, entity, e_disc,
           src_sim, dst_sim, src_pre, dst_pre, src_ec, dst_ec,
           W_sim, b_sim, W_pre, b_pre, W_ec, b_ec,
           fs_W, fs_b, fe_W, fe_b, p1_W, p1_b, p2_W, p2_b, p3_W, p3_b):
    N, K = entity.shape
    E = src_sim.shape[0]
    B, H = history.shape
    EXN = e_disc.shape[0]
    i32 = jnp.int32

    assert N % NS == 0 and B % (NC * NS) == 0
    CPT = -(-E // (NS * IC))
    CPT = -(-CPT // (2 * WV)) * (2 * WV)
    EP = NS * CPT * IC
    NP = -(-(N + 1) // IC) * IC
    HN = NP // 2
    HNA = HN + IC
    R = 544
    assert N % R == 0
    HP = -(-H // 8) * 8

    def pad_idx(a):
        return jnp.concatenate([a, jnp.full((EP - E,), N, i32)])

    def redirect(a):
        return jnp.stack([
            jnp.where((a >= c * HN) & (a < (c + 1) * HN), a - c * HN, HN)
            for c in range(NC)])

    srcp = jnp.stack([pad_idx(src_sim), pad_idx(src_pre), pad_idx(src_ec)])
    dstp = jnp.stack([pad_idx(dst_sim), pad_idx(dst_pre), pad_idx(dst_ec)])
    dstR = redirect(dstp)[:, :, None, :]
    deg_idxR = redirect(jnp.concatenate([srcp, dstp], axis=0))[:, :, None, :]

    deg = _deg_call(deg_idxR, NP=NP, HN=HN, HNA=HNA, CPT=CPT)
    rout, rin, h0, h1, h2 = _prep_call(deg, entity, NP=NP, R=R)

    facc = entity
    dacc = None
    full = disc = None
    for l in range(1, 5):
        agg = _agg_call(h0, h1, h2, srcp[:, None, :], dstR,
                        NP=NP, HN=HN, HNA=HNA, CPT=CPT)
        Wst = jnp.stack([W_sim[l - 1], W_pre[l - 1], W_ec[l - 1]])
        bs = (b_sim[l - 1] + b_pre[l - 1] + b_ec[l - 1]).reshape(1, K)
        if l == 3:
            h0, h1, h2, facc, dacc = _layer_call(
                l, agg, rin, rout, Wst, bs, facc, None, NP=NP, R=R)
        elif l < 4:
            h0, h1, h2, facc = _layer_call(
                l, agg, rin, rout, Wst, bs, facc, None, NP=NP, R=R)
        else:
            full, disc = _layer_call(
                l, agg, rin, None, Wst, bs, facc, dacc, NP=NP, R=R)

    histp = jnp.pad(history + K, ((0, 0), (0, HP - H)))[:, None, :]
    eshift = exer_id + K
    edtab = jnp.broadcast_to(e_disc, (EXN, 128))
    stu, exe, edc = _gath_call(disc, full, histp, eshift, exer_id, edtab,
                               B=B, HH=H)
    concept = full[:K]
    edisc = edc[:, 0:1]
    return _head(stu, exe, concept, edisc, kn_emb,
                 fs_W, fs_b, fe_W, fe_b, p1_W, p1_b, p2_W, p2_b, p3_W, p3_b)
